# Initial kernel scaffold; baseline (speedup 1.0000x reference)
import jax
import jax.numpy as jnp
from jax.experimental import pallas as pl


def kernel(edge_index, user_weight, item_weight):
    raise NotImplementedError("write your pallas kernel here")


# speedup vs baseline: 7.6118x; 7.6118x over previous
import functools

import jax
import jax.numpy as jnp
from jax import lax
from jax.experimental import pallas as pl
from jax.experimental.pallas import tpu as pltpu
from jax.experimental.pallas import tpu_sc as plsc

N_USERS = 25000
N_ITEMS = 25000
N = N_USERS + N_ITEMS
D = 64
E = 800000
L = 3

NPAD = 51200
EP = 819200
NCHUNK = EP // 128
TILE_CHUNKS = NCHUNK // 16
TILE_GROUPS = TILE_CHUNKS // 8
DEG_CHUNKS = NCHUNK // 32
DEG_GROUPS = DEG_CHUNKS // 8
NSL = NPAD // 16

BN = 512
NB = NPAD // BN


def _mesh():
    return plsc.VectorSubcoreMesh(core_axis_name="c", subcore_axis_name="s")


def _sc_deg(row_chunks):
    @functools.partial(
        pl.kernel,
        out_type=jax.ShapeDtypeStruct((2, NPAD), jnp.float32),
        mesh=_mesh(),
        scratch_types=[
            pltpu.VMEM((8, 128), jnp.int32),
            pltpu.VMEM((128,), jnp.float32),
            pltpu.VMEM((NSL,), jnp.float32),
            pltpu.VMEM_SHARED((NPAD,), jnp.float32),
        ],
        compiler_params=pltpu.CompilerParams(use_tc_tiling_on_sc=False),
    )
    def k(row_hbm, out_hbm, ibuf, ones, zbuf, dsp):
        c = lax.axis_index("c")
        s = lax.axis_index("s")
        one16 = jnp.ones((16,), jnp.float32)
        z16 = jnp.zeros((16,), jnp.float32)
        for i in range(8):
            ones[pl.ds(i * 16, 16)] = one16

        def zfill(i, _):
            zbuf[pl.ds(i * 16, 16)] = z16
            return 0

        lax.fori_loop(0, NSL // 16, zfill, 0)
        pltpu.sync_copy(zbuf, dsp.at[pl.ds(s * NSL, NSL)])
        plsc.subcore_barrier()

        tile0 = (c * 16 + s) * DEG_CHUNKS

        def grp(g, _):
            pltpu.sync_copy(row_hbm.at[pl.ds(tile0 + g * 8, 8)], ibuf)
            for j in range(8):
                pltpu.sync_copy(ones, dsp.at[ibuf.at[j]], add=True)
            return 0

        lax.fori_loop(0, DEG_GROUPS, grp, 0)
        plsc.subcore_barrier()
        pltpu.sync_copy(dsp.at[pl.ds(s * NSL, NSL)],
                        out_hbm.at[c, pl.ds(s * NSL, NSL)])

    return k(row_chunks)


def _sc_layer(y_flat, row2, col2):
    @functools.partial(
        pl.kernel,
        out_type=jax.ShapeDtypeStruct((2, NPAD, 32), jnp.float32),
        mesh=_mesh(),
        scratch_types=[
            pltpu.VMEM((8, 128), jnp.int32),
            pltpu.VMEM((8, 128), jnp.int32),
            pltpu.VMEM((128, 32), jnp.float32),
            pltpu.VMEM((128, 32), jnp.float32),
            pltpu.VMEM_SHARED((NPAD, 32), jnp.float32),
            pltpu.SemaphoreType.DMA,
        ],
        compiler_params=pltpu.CompilerParams(use_tc_tiling_on_sc=False),
    )
    def k(y_hbm, row_hbm, col_hbm, out_hbm, rbuf, cbuf, rows, zbuf, acc, sem):
        c = lax.axis_index("c")
        s = lax.axis_index("s")
        z16 = jnp.zeros((16,), jnp.float32)

        def zrow(i, _):
            zbuf[i, pl.ds(0, 16)] = z16
            zbuf[i, pl.ds(16, 16)] = z16
            return 0

        lax.fori_loop(0, 128, zrow, 0)

        def zacc(i, _):
            pltpu.sync_copy(zbuf, acc.at[pl.ds(s * NSL + i * 128, 128)])
            return 0

        lax.fori_loop(0, NSL // 128, zacc, 0)
        plsc.subcore_barrier()

        def grp(g, _):
            cstart = s * TILE_CHUNKS + g * 8
            pltpu.sync_copy(row_hbm.at[c, pl.ds(cstart, 8)], rbuf)
            pltpu.sync_copy(col_hbm.at[pl.ds(cstart, 8)], cbuf)
            for j in range(8):
                pltpu.async_copy(y_hbm.at[rbuf.at[j]], rows, sem).wait()
                pltpu.sync_copy(rows, acc.at[cbuf.at[j]], add=True)
            return 0

        lax.fori_loop(0, TILE_GROUPS, grp, 0)
        plsc.subcore_barrier()
        pltpu.sync_copy(acc.at[pl.ds(s * NSL, NSL)],
                        out_hbm.at[c, pl.ds(s * NSL, NSL)])

    return k(y_flat, row2, col2)


def _tc_prep(x0split, degp):
    def body(x_ref, degp_ref, y_ref, dis_ref):
        deg = degp_ref[0, :] + degp_ref[1, :]
        dis = jnp.where(deg > 0.0, lax.rsqrt(deg), 0.0)
        dis_ref[...] = dis
        y_ref[0] = x_ref[0] * dis[:, None]

    return pl.pallas_call(
        body,
        grid=(2, NB),
        in_specs=[
            pl.BlockSpec((1, BN, 32), lambda c, i: (c, i, 0)),
            pl.BlockSpec((2, BN), lambda c, i: (0, i)),
        ],
        out_specs=[
            pl.BlockSpec((1, BN, 32), lambda c, i: (c, i, 0)),
            pl.BlockSpec((BN,), lambda c, i: (i,)),
        ],
        out_shape=[
            jax.ShapeDtypeStruct((2, NPAD, 32), jnp.float32),
            jax.ShapeDtypeStruct((NPAD,), jnp.float32),
        ],
    )(x0split, degp)


def _tc_scale(acc, dis):
    def body(a_ref, dis_ref, y_ref):
        d = dis_ref[...]
        y_ref[0] = a_ref[0] * (d * d)[:, None]

    return pl.pallas_call(
        body,
        grid=(2, NB),
        in_specs=[
            pl.BlockSpec((1, BN, 32), lambda c, i: (c, i, 0)),
            pl.BlockSpec((BN,), lambda c, i: (i,)),
        ],
        out_specs=pl.BlockSpec((1, BN, 32), lambda c, i: (c, i, 0)),
        out_shape=jax.ShapeDtypeStruct((2, NPAD, 32), jnp.float32),
    )(acc, dis)


def _tc_epilogue(x0split, a0, a1, a2, dis):
    def body(x_ref, a0_ref, a1_ref, a2_ref, dis_ref, f_ref):
        ssum = a0_ref[0] + a1_ref[0] + a2_ref[0]
        d = dis_ref[...]
        f_ref[0] = 0.25 * x_ref[0] + 0.25 * d[:, None] * ssum

    acc_spec = pl.BlockSpec((1, BN, 32), lambda c, i: (c, i, 0))
    return pl.pallas_call(
        body,
        grid=(2, NB),
        in_specs=[
            acc_spec,
            acc_spec, acc_spec, acc_spec,
            pl.BlockSpec((BN,), lambda c, i: (i,)),
        ],
        out_specs=pl.BlockSpec((1, BN, 32), lambda c, i: (c, i, 0)),
        out_shape=jax.ShapeDtypeStruct((2, NPAD, 32), jnp.float32),
    )(x0split, a0, a1, a2, dis)


def kernel(edge_index, user_weight, item_weight):
    row = edge_index[0]
    col = edge_index[1]
    pad = jnp.full((EP - E,), N, dtype=jnp.int32)
    rowp = jnp.concatenate([row, pad])
    colp = jnp.concatenate([col, pad])
    row2 = jnp.stack([rowp, rowp + NPAD]).reshape(2, NCHUNK, 128)
    col2 = colp.reshape(NCHUNK, 128)
    rowc = rowp.reshape(NCHUNK, 128)

    x0 = jnp.concatenate([user_weight, item_weight], axis=0)
    x0pad = jnp.concatenate(
        [x0, jnp.zeros((NPAD - N, D), jnp.float32)], axis=0)
    x0split = x0pad.reshape(NPAD, 2, 32).transpose(1, 0, 2)

    degp = _sc_deg(rowc)
    y0, dis = _tc_prep(x0split, degp)

    accs = []
    y = y0
    for l in range(L):
        acc = _sc_layer(y.reshape(2 * NPAD, 32), row2, col2)
        accs.append(acc)
        if l < L - 1:
            y = _tc_scale(acc, dis)

    fin = _tc_epilogue(x0split, accs[0], accs[1], accs[2], dis)
    out = fin[:, :N, :].transpose(1, 0, 2).reshape(N, D)
    return out[:N_USERS], out[N_USERS:]
